# baseline (speedup 1.0000x reference)
import functools

import jax
import jax.numpy as jnp
from jax import lax
from jax.experimental import pallas as pl
from jax.experimental.pallas import tpu as pltpu
from jax.experimental.pallas import tpu_sc as plsc

VOCAB = 1000000
EMBED_DIM = 32
BATCH = 4096
SEQ = 200

_INFO = plsc.get_sparse_core_info()
NC, NS, NL = _INFO.num_cores, _INFO.num_subcores, _INFO.num_lanes
NW = NC * NS
B_PER_W = BATCH // NW
LG = 8
N_GROUPS = SEQ // LG
N_PAIRS = (N_GROUPS - 1) // 2
SUBCOLS = B_PER_W // NL


def _body(x_h, emb_h, out_h, x_v, idx0, idx1, buf0, buf1,
          gsem0, gsem1, wsem0, wsem1):
    wid = lax.axis_index("s") * NC + lax.axis_index("c")
    b_base = wid * B_PER_W
    pltpu.sync_copy(x_h.at[pl.ds(b_base, B_PER_W), :], x_v)

    lane = lax.iota(jnp.int32, NL)

    def build_idx(g, idx_v):
        for b in range(LG):
            col = jnp.broadcast_to(g * LG + b, (NL,)).astype(jnp.int32)
            for j in range(SUBCOLS):
                rows = lane + (j * NL)
                vals = plsc.load_gather(x_v, [rows, col])
                idx_v[b, pl.ds(j * NL, NL)] = vals

    def fire(idx_v, buf, sem):
        for b in range(LG):
            pltpu.async_copy(emb_h.at[idx_v.at[b]], buf.at[b], sem)

    def drain_gather(buf, sem):
        pltpu.make_async_copy(
            out_h.at[pl.ds(0, LG), pl.ds(0, B_PER_W), :], buf, sem).wait()

    def writeback(g, buf, sem):
        pltpu.async_copy(
            buf, out_h.at[pl.ds(g * LG, LG), pl.ds(b_base, B_PER_W), :], sem)

    def wait_writeback(buf, sem):
        pltpu.make_async_copy(
            buf, out_h.at[pl.ds(0, LG), pl.ds(0, B_PER_W), :], sem).wait()

    def group(g, idx_v, buf, gsem, wsem, first):
        if not first:
            wait_writeback(buf, wsem)
        build_idx(g, idx_v)
        fire(idx_v, buf, gsem)

    group(0, idx0, buf0, gsem0, wsem0, first=True)
    group(1, idx1, buf1, gsem1, wsem1, first=True)

    def body(i, carry):
        g0 = 2 * i
        drain_gather(buf0, gsem0)
        writeback(g0, buf0, wsem0)
        drain_gather(buf1, gsem1)
        writeback(g0 + 1, buf1, wsem1)

        wait_writeback(buf0, wsem0)
        build_idx(g0 + 2, idx0)
        fire(idx0, buf0, gsem0)

        @pl.when(i < N_PAIRS - 1)
        def _():
            wait_writeback(buf1, wsem1)
            build_idx(g0 + 3, idx1)
            fire(idx1, buf1, gsem1)

        return carry

    lax.fori_loop(0, N_PAIRS, body, 0)

    drain_gather(buf0, gsem0)
    writeback(N_GROUPS - 1, buf0, wsem0)
    wait_writeback(buf0, wsem0)
    wait_writeback(buf1, wsem1)


@functools.partial(jax.jit, static_argnames=())
def kernel(x, hidden, emb):
    del hidden
    mesh = plsc.VectorSubcoreMesh(core_axis_name="c", subcore_axis_name="s")
    return pl.kernel(
        _body,
        mesh=mesh,
        out_type=jax.ShapeDtypeStruct((SEQ, BATCH, EMBED_DIM), jnp.float32),
        scratch_types=[
            pltpu.VMEM((B_PER_W, SEQ), jnp.int32),
            pltpu.VMEM((LG, B_PER_W), jnp.int32),
            pltpu.VMEM((LG, B_PER_W), jnp.int32),
            pltpu.VMEM((LG, B_PER_W, EMBED_DIM), jnp.float32),
            pltpu.VMEM((LG, B_PER_W, EMBED_DIM), jnp.float32),
            pltpu.SemaphoreType.DMA,
            pltpu.SemaphoreType.DMA,
            pltpu.SemaphoreType.DMA,
            pltpu.SemaphoreType.DMA,
        ],
        compiler_params=pltpu.CompilerParams(
            use_tc_tiling_on_sc=False, needs_layout_passes=False),
    )(x, emb)


# speedup vs baseline: 1.3265x; 1.3265x over previous
import functools

import jax
import jax.numpy as jnp
from jax import lax
from jax.experimental import pallas as pl
from jax.experimental.pallas import tpu as pltpu
from jax.experimental.pallas import tpu_sc as plsc

VOCAB = 1000000
EMBED_DIM = 32
BATCH = 4096
SEQ = 200

_INFO = plsc.get_sparse_core_info()
NC, NS, NL = _INFO.num_cores, _INFO.num_subcores, _INFO.num_lanes
NW = NC * NS
BT = BATCH // NW
LG = 4
N_GROUPS = SEQ // LG
N_PAIRS = N_GROUPS // 2
DHI = EMBED_DIM // 8
INNER = 8 * BT


def _body(x_h, emb_h, out_h, x_v, rows0, rows1, tb0, tb1,
          gsem0, gsem1, wsem0, wsem1):
    wid = lax.axis_index("s") * NC + lax.axis_index("c")
    pltpu.sync_copy(x_h.at[:, wid], x_v)

    iota = lax.iota(jnp.int32, NL)

    def fire(g, rows, sem):
        for li in range(LG):
            l = g * LG + li
            pltpu.async_copy(
                emb_h.at[x_v.at[l // 8, l % 8]], rows.at[li], sem)

    def drain_gather(rows, sem):
        for li in range(LG):
            pltpu.make_async_copy(
                emb_h.at[pl.ds(0, BT)], rows.at[li], sem).wait()

    def transpose_group(rows, tb):
        for li in range(LG):
            li_vec = jnp.broadcast_to(jnp.int32(li), (NL,))

            def tbody(t, carry):
                p = (iota + t) & 15
                pats = []
                for dblk in range(2):
                    patd = p + dblk * 16
                    pats.append((patd, patd >> 3, (patd & 7) << 7))
                for c in range(BT // NL):
                    b = iota + c * NL
                    for dblk in range(2):
                        patd, pathi, patflat = pats[dblk]
                        vals = plsc.load_gather(rows, [li_vec, b, patd])
                        plsc.store_scatter(
                            tb, [li_vec, pathi, patflat + b], vals)
                return carry

            lax.fori_loop(0, 16, tbody, 0)

    def writeback(g, tb, sem):
        pltpu.async_copy(
            tb, out_h.at[pl.ds(g * LG, LG), :, pl.ds(wid * INNER, INNER)],
            sem)

    def wait_writeback(tb, sem):
        pltpu.make_async_copy(
            tb, out_h.at[pl.ds(0, LG), :, pl.ds(0, INNER)], sem).wait()

    fire(0, rows0, gsem0)
    fire(1, rows1, gsem1)

    def body(i, carry):
        g0 = 2 * i

        drain_gather(rows0, gsem0)

        @pl.when(i > 0)
        def _():
            wait_writeback(tb0, wsem0)

        transpose_group(rows0, tb0)
        writeback(g0, tb0, wsem0)

        @pl.when(i < N_PAIRS - 1)
        def _():
            fire(g0 + 2, rows0, gsem0)

        drain_gather(rows1, gsem1)

        @pl.when(i > 0)
        def _():
            wait_writeback(tb1, wsem1)

        transpose_group(rows1, tb1)
        writeback(g0 + 1, tb1, wsem1)

        @pl.when(i < N_PAIRS - 1)
        def _():
            fire(g0 + 3, rows1, gsem1)

        return carry

    lax.fori_loop(0, N_PAIRS, body, 0)
    wait_writeback(tb0, wsem0)
    wait_writeback(tb1, wsem1)


@functools.partial(jax.jit, static_argnames=())
def kernel(x, hidden, emb):
    del hidden
    x5 = jnp.transpose(x, (1, 0)).reshape(SEQ // 8, 8, NW, BT)
    x5 = jnp.transpose(x5, (0, 2, 1, 3))

    mesh = plsc.VectorSubcoreMesh(core_axis_name="c", subcore_axis_name="s")
    y3 = pl.kernel(
        _body,
        mesh=mesh,
        out_type=jax.ShapeDtypeStruct((SEQ, DHI, NW * INNER), jnp.float32),
        scratch_types=[
            pltpu.VMEM((SEQ // 8, 8, BT), jnp.int32),
            pltpu.VMEM((LG, BT, EMBED_DIM), jnp.float32),
            pltpu.VMEM((LG, BT, EMBED_DIM), jnp.float32),
            pltpu.VMEM((LG, DHI, INNER), jnp.float32),
            pltpu.VMEM((LG, DHI, INNER), jnp.float32),
            pltpu.SemaphoreType.DMA,
            pltpu.SemaphoreType.DMA,
            pltpu.SemaphoreType.DMA,
            pltpu.SemaphoreType.DMA,
        ],
        compiler_params=pltpu.CompilerParams(
            use_tc_tiling_on_sc=False, needs_layout_passes=False),
    )(x5, emb)

    y5 = y3.reshape(SEQ, DHI, NW, 8, BT)
    return jnp.transpose(y5, (0, 2, 4, 1, 3)).reshape(SEQ, BATCH, EMBED_DIM)


# speedup vs baseline: 1.3501x; 1.0178x over previous
import functools

import jax
import jax.numpy as jnp
from jax import lax
from jax.experimental import pallas as pl
from jax.experimental.pallas import tpu as pltpu
from jax.experimental.pallas import tpu_sc as plsc

VOCAB = 1000000
EMBED_DIM = 32
BATCH = 4096
SEQ = 200

_INFO = plsc.get_sparse_core_info()
NC, NS, NL = _INFO.num_cores, _INFO.num_subcores, _INFO.num_lanes
NW = NC * NS
BT = BATCH // NW
LG = 4
N_GROUPS = SEQ // LG
NRB = 3
NTB = 2
UNROLL = NRB * NTB
N_MAIN = (N_GROUPS - 2) // UNROLL
DHI = EMBED_DIM // 8
INNER = 8 * BT


def _body(x_h, emb_h, out_h, x_v, rows0, rows1, rows2, tb0, tb1,
          gsem0, gsem1, gsem2, wsem0, wsem1):
    rows = (rows0, rows1, rows2)
    gsem = (gsem0, gsem1, gsem2)
    tb = (tb0, tb1)
    wsem = (wsem0, wsem1)

    wid = lax.axis_index("s") * NC + lax.axis_index("c")
    pltpu.sync_copy(x_h.at[:, wid], x_v)

    iota = lax.iota(jnp.int32, NL)
    li_vecs = [jnp.broadcast_to(jnp.int32(li), (NL,)) for li in range(LG)]

    def fire(g, r, sem):
        for li in range(LG):
            l = g * LG + li
            pltpu.async_copy(
                emb_h.at[x_v.at[l // 8, l % 8]], r.at[li], sem)

    def drain_gather(r, sem):
        for li in range(LG):
            pltpu.make_async_copy(
                emb_h.at[pl.ds(0, BT)], r.at[li], sem).wait()

    def transpose_group(r, t):
        def tbody(tt, carry):
            p = (iota + tt) & 15
            pats = []
            for dblk in range(2):
                patd = p + dblk * 16
                pats.append((patd, patd >> 3, (patd & 7) << 7))
            for li in range(LG):
                for c in range(BT // NL):
                    b = iota + c * NL
                    for dblk in range(2):
                        patd, pathi, patflat = pats[dblk]
                        vals = plsc.load_gather(r, [li_vecs[li], b, patd])
                        plsc.store_scatter(
                            t, [li_vecs[li], pathi, patflat + b], vals)
            return carry

        lax.fori_loop(0, 16, tbody, 0)

    def writeback(g, t, sem):
        pltpu.async_copy(
            t, out_h.at[pl.ds(g * LG, LG), :, pl.ds(wid * INNER, INNER)],
            sem)

    def wait_writeback(t, sem):
        pltpu.make_async_copy(
            t, out_h.at[pl.ds(0, LG), :, pl.ds(0, INNER)], sem).wait()

    def handle_group(g, ri, ti, guard_wb, guard_fire):
        drain_gather(rows[ri], gsem[ri])
        if guard_wb is None:
            wait_writeback(tb[ti], wsem[ti])
        elif guard_wb is not False:
            @pl.when(guard_wb)
            def _():
                wait_writeback(tb[ti], wsem[ti])
        transpose_group(rows[ri], tb[ti])
        writeback(g, tb[ti], wsem[ti])
        if guard_fire is None:
            fire(g + NRB, rows[ri], gsem[ri])
        elif guard_fire is not False:
            @pl.when(guard_fire)
            def _():
                fire(g + NRB, rows[ri], gsem[ri])

    for g0 in range(NRB):
        fire(g0, rows[g0], gsem[g0])

    def body(i, carry):
        gb = UNROLL * i
        for j in range(UNROLL):
            g = gb + j
            handle_group(g, j % NRB, j % NTB,
                         guard_wb=(g >= NTB) if j < NTB else None,
                         guard_fire=(g + NRB < N_GROUPS))
        return carry

    lax.fori_loop(0, N_MAIN, body, 0)

    for g in range(N_MAIN * UNROLL, N_GROUPS):
        handle_group(g, g % NRB, g % NTB, guard_wb=None, guard_fire=False)

    wait_writeback(tb[0], wsem[0])
    wait_writeback(tb[1], wsem[1])


@functools.partial(jax.jit, static_argnames=())
def kernel(x, hidden, emb):
    del hidden
    x5 = jnp.transpose(x, (1, 0)).reshape(SEQ // 8, 8, NW, BT)
    x5 = jnp.transpose(x5, (0, 2, 1, 3))

    mesh = plsc.VectorSubcoreMesh(core_axis_name="c", subcore_axis_name="s")
    y3 = pl.kernel(
        _body,
        mesh=mesh,
        out_type=jax.ShapeDtypeStruct((SEQ, DHI, NW * INNER), jnp.float32),
        scratch_types=[
            pltpu.VMEM((SEQ // 8, 8, BT), jnp.int32),
            pltpu.VMEM((LG, BT, EMBED_DIM), jnp.float32),
            pltpu.VMEM((LG, BT, EMBED_DIM), jnp.float32),
            pltpu.VMEM((LG, BT, EMBED_DIM), jnp.float32),
            pltpu.VMEM((LG, DHI, INNER), jnp.float32),
            pltpu.VMEM((LG, DHI, INNER), jnp.float32),
            pltpu.SemaphoreType.DMA,
            pltpu.SemaphoreType.DMA,
            pltpu.SemaphoreType.DMA,
            pltpu.SemaphoreType.DMA,
            pltpu.SemaphoreType.DMA,
        ],
        compiler_params=pltpu.CompilerParams(
            use_tc_tiling_on_sc=False, needs_layout_passes=False),
    )(x5, emb)

    y5 = y3.reshape(SEQ, DHI, NW, 8, BT)
    return jnp.transpose(y5, (0, 2, 4, 1, 3)).reshape(SEQ, BATCH, EMBED_DIM)


# speedup vs baseline: 1.6294x; 1.2069x over previous
import functools

import jax
import jax.numpy as jnp
from jax import lax
from jax.experimental import pallas as pl
from jax.experimental.pallas import tpu as pltpu
from jax.experimental.pallas import tpu_sc as plsc

VOCAB = 1000000
EMBED_DIM = 32
BATCH = 4096
SEQ = 200

_INFO = plsc.get_sparse_core_info()
NC, NS, NL = _INFO.num_cores, _INFO.num_subcores, _INFO.num_lanes
NW = NC * NS
BT = BATCH // NW
LG = 4
N_GROUPS = SEQ // LG
NRB = 3
NTB = 2
UNROLL = NRB * NTB
N_MAIN = (N_GROUPS - 2) // UNROLL
DHI = EMBED_DIM // 8
INNER = 8 * BT


def _body(x_h, emb_h, out_h, x_v, rows0, rows1, rows2, tb0, tb1,
          gsem0, gsem1, gsem2, wsem0, wsem1):
    rows = (rows0, rows1, rows2)
    gsem = (gsem0, gsem1, gsem2)
    tb = (tb0, tb1)
    wsem = (wsem0, wsem1)

    wid = lax.axis_index("s") * NC + lax.axis_index("c")
    pltpu.sync_copy(x_h.at[:, wid], x_v)

    iota = lax.iota(jnp.int32, NL)
    li_vecs = [jnp.broadcast_to(jnp.int32(li), (NL,)) for li in range(LG)]

    def fire(g, r, sem):
        for li in range(LG):
            l = g * LG + li
            pltpu.async_copy(
                emb_h.at[x_v.at[l // 8, l % 8]], r.at[li], sem)

    def drain_gather(r, sem):
        for li in range(LG):
            pltpu.make_async_copy(
                emb_h.at[pl.ds(0, BT)], r.at[li], sem).wait()

    def transpose_group(r, t):
        @plsc.parallel_loop(0, 16, unroll=2)
        def tbody(tt):
            p = (iota + tt) & 15
            pats = []
            for dblk in range(2):
                patd = p + dblk * 16
                pats.append((patd, patd >> 3, (patd & 7) << 7))
            for li in range(LG):
                staged = []
                for c in range(BT // NL):
                    b = iota + c * NL
                    for dblk in range(2):
                        patd, pathi, patflat = pats[dblk]
                        staged.append(
                            (plsc.load_gather(r, [li_vecs[li], b, patd]),
                             pathi, patflat + b))
                for vals, pathi, flat in staged:
                    plsc.store_scatter(t, [li_vecs[li], pathi, flat], vals)

    def writeback(g, t, sem):
        pltpu.async_copy(
            t, out_h.at[pl.ds(g * LG, LG), :, pl.ds(wid * INNER, INNER)],
            sem)

    def wait_writeback(t, sem):
        pltpu.make_async_copy(
            t, out_h.at[pl.ds(0, LG), :, pl.ds(0, INNER)], sem).wait()

    def handle_group(g, ri, ti, guard_wb, guard_fire):
        drain_gather(rows[ri], gsem[ri])
        if guard_wb is None:
            wait_writeback(tb[ti], wsem[ti])
        elif guard_wb is not False:
            @pl.when(guard_wb)
            def _():
                wait_writeback(tb[ti], wsem[ti])
        transpose_group(rows[ri], tb[ti])
        writeback(g, tb[ti], wsem[ti])
        if guard_fire is None:
            fire(g + NRB, rows[ri], gsem[ri])
        elif guard_fire is not False:
            @pl.when(guard_fire)
            def _():
                fire(g + NRB, rows[ri], gsem[ri])

    for g0 in range(NRB):
        fire(g0, rows[g0], gsem[g0])

    def body(i, carry):
        gb = UNROLL * i
        for j in range(UNROLL):
            g = gb + j
            handle_group(g, j % NRB, j % NTB,
                         guard_wb=(g >= NTB) if j < NTB else None,
                         guard_fire=(g + NRB < N_GROUPS))
        return carry

    lax.fori_loop(0, N_MAIN, body, 0)

    for g in range(N_MAIN * UNROLL, N_GROUPS):
        handle_group(g, g % NRB, g % NTB, guard_wb=None, guard_fire=False)

    wait_writeback(tb[0], wsem[0])
    wait_writeback(tb[1], wsem[1])


@functools.partial(jax.jit, static_argnames=())
def kernel(x, hidden, emb):
    del hidden
    x5 = jnp.transpose(x, (1, 0)).reshape(SEQ // 8, 8, NW, BT)
    x5 = jnp.transpose(x5, (0, 2, 1, 3))

    mesh = plsc.VectorSubcoreMesh(core_axis_name="c", subcore_axis_name="s")
    y3 = pl.kernel(
        _body,
        mesh=mesh,
        out_type=jax.ShapeDtypeStruct((SEQ, DHI, NW * INNER), jnp.float32),
        scratch_types=[
            pltpu.VMEM((SEQ // 8, 8, BT), jnp.int32),
            pltpu.VMEM((LG, BT, EMBED_DIM), jnp.float32),
            pltpu.VMEM((LG, BT, EMBED_DIM), jnp.float32),
            pltpu.VMEM((LG, BT, EMBED_DIM), jnp.float32),
            pltpu.VMEM((LG, DHI, INNER), jnp.float32),
            pltpu.VMEM((LG, DHI, INNER), jnp.float32),
            pltpu.SemaphoreType.DMA,
            pltpu.SemaphoreType.DMA,
            pltpu.SemaphoreType.DMA,
            pltpu.SemaphoreType.DMA,
            pltpu.SemaphoreType.DMA,
        ],
        compiler_params=pltpu.CompilerParams(
            use_tc_tiling_on_sc=False, needs_layout_passes=False),
    )(x5, emb)

    y5 = y3.reshape(SEQ, DHI, NW, 8, BT)
    return jnp.transpose(y5, (0, 2, 4, 1, 3)).reshape(SEQ, BATCH, EMBED_DIM)
